# Initial kernel scaffold; baseline (speedup 1.0000x reference)
import jax
import jax.numpy as jnp
from jax.experimental import pallas as pl


def kernel(x, W_shared, b_shared, W_experts, b_experts, W_router, b_router):
    raise NotImplementedError("write your pallas kernel here")


# speedup vs baseline: 1.9744x; 1.9744x over previous
import functools

import jax
import jax.numpy as jnp
from jax.experimental import pallas as pl

NUM_EXPERTS = 8
INPUT_DIM = 768
HIDDEN_DIM = 768
NUM_TOKENS = 32768

BT = 512


def _moe_block_kernel(xf_ref, x_ref, ws_ref, we_ref, br_ref, wr_ref, bs_ref,
                      be_ref, out_ref):
    x = x_ref[...]
    logits = jnp.dot(xf_ref[...], wr_ref[...],
                     preferred_element_type=jnp.float32)
    logits = logits + br_ref[...]
    e = jnp.argmax(logits, axis=-1)
    acc = jnp.dot(x, ws_ref[...], preferred_element_type=jnp.float32)
    acc = acc + bs_ref[...]
    for i in range(NUM_EXPERTS):
        mask = (e == i).astype(jnp.float32)[:, None]
        eo = jnp.dot(x, we_ref[i], preferred_element_type=jnp.float32)
        eo = eo + be_ref[i][None, :]
        acc = acc + mask * eo
    out_ref[...] = acc


def kernel(x, W_shared, b_shared, W_experts, b_experts, W_router, b_router):
    n = x.shape[0]
    xb = x.astype(jnp.bfloat16)
    wsb = W_shared.astype(jnp.bfloat16)
    web = W_experts.astype(jnp.bfloat16)
    grid = (n // BT,)
    out = pl.pallas_call(
        _moe_block_kernel,
        grid=grid,
        in_specs=[
            pl.BlockSpec((BT, INPUT_DIM), lambda b: (b, 0)),
            pl.BlockSpec((BT, INPUT_DIM), lambda b: (b, 0)),
            pl.BlockSpec((INPUT_DIM, HIDDEN_DIM), lambda b: (0, 0)),
            pl.BlockSpec((NUM_EXPERTS, INPUT_DIM, HIDDEN_DIM),
                         lambda b: (0, 0, 0)),
            pl.BlockSpec((1, NUM_EXPERTS), lambda b: (0, 0)),
            pl.BlockSpec((INPUT_DIM, NUM_EXPERTS), lambda b: (0, 0)),
            pl.BlockSpec((1, HIDDEN_DIM), lambda b: (0, 0)),
            pl.BlockSpec((NUM_EXPERTS, HIDDEN_DIM), lambda b: (0, 0)),
        ],
        out_specs=pl.BlockSpec((BT, HIDDEN_DIM), lambda b: (b, 0)),
        out_shape=jax.ShapeDtypeStruct((n, HIDDEN_DIM), jnp.float32),
    )(x, xb, wsb, web, b_router.reshape(1, -1), W_router,
      b_shared.reshape(1, -1), b_experts)
    return out
